# baseline (speedup 1.0000x reference)
import functools

import jax
import jax.numpy as jnp
from jax import lax
from jax.experimental import pallas as pl
from jax.experimental.pallas import tpu as pltpu
from jax.experimental.pallas import tpu_sc as plsc

N_NODES = 10000
NPAD = 10240
IN_DIM = 128
HID_DIM = 128
OUT_DIM = 64
N_EDGES = 320000

NC = 2
NS = 16
NW = NC * NS
EPW = 10240
E_PAD = NW * EPW
K = 80
CH = EPW // K
IB = 8
NB = CH // IB
RPT = NPAD // NS

_SELU_ALPHA = 1.6732632423543772
_SELU_SCALE = 1.0507009873554805


def _mesh():
    return plsc.VectorSubcoreMesh(core_axis_name="c", subcore_axis_name="s")


def _sc_degree(dst_r):
    @functools.partial(
        pl.kernel,
        out_type=jax.ShapeDtypeStruct((NC, NPAD), jnp.float32),
        mesh=_mesh(),
        scratch_types=[
            pltpu.VMEM((CH, K), jnp.int32),
            pltpu.VMEM((K,), jnp.float32),
            pltpu.VMEM((RPT,), jnp.float32),
            pltpu.VMEM_SHARED((NPAD,), jnp.float32),
        ],
    )
    def deg_kernel(dst_hbm, out_hbm, dstv, onesv, zv, acc_sh):
        c = lax.axis_index("c")
        s = lax.axis_index("s")
        wid = s * NC + c

        def fill(i, _):
            zv[pl.ds(i * 16, 16)] = jnp.zeros((16,), jnp.float32)
            return 0

        lax.fori_loop(0, RPT // 16, fill, 0)
        for i in range(K // 16):
            onesv[pl.ds(i * 16, 16)] = jnp.ones((16,), jnp.float32)
        pltpu.sync_copy(zv, acc_sh.at[pl.ds(s * RPT, RPT)])
        pltpu.sync_copy(dst_hbm.at[wid], dstv)
        plsc.subcore_barrier()

        def body(j, _):
            pltpu.sync_copy(onesv, acc_sh.at[dstv.at[j]], add=True)
            return 0

        lax.fori_loop(0, CH, body, 0)
        plsc.subcore_barrier()
        pltpu.sync_copy(acc_sh.at[pl.ds(s * RPT, RPT)],
                        out_hbm.at[c, pl.ds(s * RPT, RPT)])

    return deg_kernel(dst_r)


def _sc_scatter(hs, src_r, dst_r, d, tc_tiling=True):
    @functools.partial(
        pl.kernel,
        out_type=jax.ShapeDtypeStruct((NC, NPAD, d), jnp.float32),
        mesh=_mesh(),
        compiler_params=pltpu.CompilerParams(use_tc_tiling_on_sc=tc_tiling),
        scratch_types=[
            pltpu.VMEM((2, IB, K), jnp.int32),
            pltpu.VMEM((2, IB, K), jnp.int32),
            pltpu.VMEM((K, d), jnp.float32),
            pltpu.VMEM((K, d), jnp.float32),
            pltpu.VMEM_SHARED((NPAD, d), jnp.float32),
            pltpu.SemaphoreType.DMA,
            pltpu.SemaphoreType.DMA,
            pltpu.SemaphoreType.DMA,
            pltpu.SemaphoreType.DMA,
            pltpu.SemaphoreType.DMA,
        ],
    )
    def scat_kernel(hs_hbm, src_hbm, dst_hbm, out_hbm,
                    sib, dib, rows0, rows1, acc_sh,
                    sem0, sem1, semsc0, semsc1, semi):
        c = lax.axis_index("c")
        s = lax.axis_index("s")
        wid = s * NC + c
        rows = (rows0, rows1)
        sems = (sem0, sem1)
        semsc = (semsc0, semsc1)

        def zfill(r, _):
            for i in range(d // 16):
                rows0[r, pl.ds(i * 16, 16)] = jnp.zeros((16,), jnp.float32)
            return 0

        lax.fori_loop(0, K, zfill, 0)
        for i in range(RPT // K):
            pltpu.sync_copy(rows0, acc_sh.at[pl.ds(s * RPT + i * K, K)])
        pltpu.sync_copy(src_hbm.at[wid, 0], sib.at[0])
        pltpu.sync_copy(dst_hbm.at[wid, 0], dib.at[0])
        plsc.subcore_barrier()
        pltpu.async_copy(hs_hbm.at[sib.at[0, 0]], rows0, sem0)

        def body(b, _):
            p = lax.rem(b, 2)

            @pl.when(b < NB - 1)
            def _():
                pltpu.async_copy(src_hbm.at[wid, b + 1], sib.at[1 - p], semi)
                pltpu.async_copy(dst_hbm.at[wid, b + 1], dib.at[1 - p], semi)

            for ch in range(IB):
                g = ch % 2
                pltpu.make_async_copy(hs_hbm.at[sib.at[p, ch]],
                                      rows[g], sems[g]).wait()
                pltpu.async_copy(rows[g], acc_sh.at[dib.at[p, ch]],
                                 semsc[g], add=True)
                if ch == 0:
                    @pl.when(b > 0)
                    def _():
                        pltpu.make_async_copy(
                            rows1, acc_sh.at[dib.at[1 - p, IB - 1]],
                            semsc[1]).wait()
                else:
                    pltpu.make_async_copy(
                        rows[1 - g], acc_sh.at[dib.at[p, ch - 1]],
                        semsc[1 - g]).wait()
                if ch + 1 < IB:
                    pltpu.async_copy(hs_hbm.at[sib.at[p, ch + 1]],
                                     rows[(ch + 1) % 2], sems[(ch + 1) % 2])
                else:
                    @pl.when(b < NB - 1)
                    def _():
                        pltpu.make_async_copy(src_hbm.at[wid, b + 1],
                                              sib.at[1 - p], semi).wait()
                        pltpu.make_async_copy(dst_hbm.at[wid, b + 1],
                                              dib.at[1 - p], semi).wait()
                        pltpu.async_copy(hs_hbm.at[sib.at[1 - p, 0]],
                                         rows0, sem0)
            return 0

        lax.fori_loop(0, NB, body, 0)
        pltpu.make_async_copy(rows1, acc_sh.at[dib.at[(NB - 1) % 2, IB - 1]],
                              semsc[1]).wait()
        plsc.subcore_barrier()
        pltpu.sync_copy(acc_sh.at[pl.ds(s * RPT, RPT)],
                        out_hbm.at[c, pl.ds(s * RPT, RPT)])

    return scat_kernel(hs, src_r, dst_r)


def _tc1_body(x_ref, w_ref, degt_ref, hs_ref, dinv_ref):
    deg = degt_ref[:N_NODES, 0:1] + degt_ref[:N_NODES, 1:2] + 1.0
    dinv = lax.rsqrt(deg)
    h = jnp.dot(x_ref[...], w_ref[...], preferred_element_type=jnp.float32)
    hs_ref[...] = dinv * h
    dinv_ref[...] = dinv


def _tc2_body(p_ref, hs1_ref, dinv_ref, b1_ref, w2_ref, hs2_ref):
    dinv = dinv_ref[...]
    z = dinv * (p_ref[0, :N_NODES, :] + p_ref[1, :N_NODES, :] + hs1_ref[...])
    z = z + b1_ref[...]
    a = _SELU_SCALE * jnp.where(z > 0, z, _SELU_ALPHA * (jnp.exp(z) - 1.0))
    h2 = jnp.dot(a, w2_ref[...], preferred_element_type=jnp.float32)
    hs2_ref[...] = dinv * h2


def _tc3_body(q_ref, hs2_ref, dinv_ref, b2_ref, out_ref):
    z = dinv_ref[...] * (q_ref[0, :N_NODES, :] + q_ref[1, :N_NODES, :]
                         + hs2_ref[...])
    out_ref[...] = z + b2_ref[...]


def kernel(x, edge_index, W1, b1, W2, b2):
    ei = edge_index.astype(jnp.int32)
    npadding = E_PAD - N_EDGES
    pad_src = (jnp.arange(npadding, dtype=jnp.int32) * 13) % N_NODES
    pad_dst = N_NODES + (jnp.arange(npadding, dtype=jnp.int32)
                         % (NPAD - N_NODES))
    src_full = jnp.concatenate([ei[0], pad_src])
    dst_full = jnp.concatenate([ei[1], pad_dst])
    dst_r = dst_full.reshape(NW, CH, K)
    src_rb = src_full.reshape(NW, NB, IB, K)
    dst_rb = dst_full.reshape(NW, NB, IB, K)

    deg_p = _sc_degree(dst_r)
    degt = deg_p.T

    hs1, dinv = pl.pallas_call(
        _tc1_body,
        out_shape=(jax.ShapeDtypeStruct((N_NODES, HID_DIM), jnp.float32),
                   jax.ShapeDtypeStruct((N_NODES, 1), jnp.float32)),
    )(x, W1, degt)

    p = _sc_scatter(hs1, src_rb, dst_rb, HID_DIM)

    hs2 = pl.pallas_call(
        _tc2_body,
        out_shape=jax.ShapeDtypeStruct((N_NODES, OUT_DIM), jnp.float32),
    )(p, hs1, dinv, b1.reshape(1, HID_DIM), W2)

    q = _sc_scatter(hs2, src_rb, dst_rb, OUT_DIM, tc_tiling=False)

    out = pl.pallas_call(
        _tc3_body,
        out_shape=jax.ShapeDtypeStruct((N_NODES, OUT_DIM), jnp.float32),
    )(q, hs2, dinv, b2.reshape(1, OUT_DIM))
    return out


# speedup vs baseline: 1.2955x; 1.2955x over previous
import functools

import jax
import jax.numpy as jnp
from jax import lax
from jax.experimental import pallas as pl
from jax.experimental.pallas import tpu as pltpu
from jax.experimental.pallas import tpu_sc as plsc

N_NODES = 10000
NPAD = 10240
IN_DIM = 128
HID_DIM = 128
OUT_DIM = 64
N_EDGES = 320000

NC = 2
NS = 16
NW = NC * NS
EPW = 10368
E_PAD = NW * EPW
K = 64
CH = 162
IB = 6
NB = CH // IB
RPT = NPAD // NS

_SELU_ALPHA = 1.6732632423543772
_SELU_SCALE = 1.0507009873554805


def _mesh():
    return plsc.VectorSubcoreMesh(core_axis_name="c", subcore_axis_name="s")


def _sc_degree(dst_r):
    @functools.partial(
        pl.kernel,
        out_type=jax.ShapeDtypeStruct((NC, NPAD), jnp.float32),
        mesh=_mesh(),
        scratch_types=[
            pltpu.VMEM((CH, K), jnp.int32),
            pltpu.VMEM((K,), jnp.float32),
            pltpu.VMEM((RPT,), jnp.float32),
            pltpu.VMEM_SHARED((NPAD,), jnp.float32),
        ],
    )
    def deg_kernel(dst_hbm, out_hbm, dstv, onesv, zv, acc_sh):
        c = lax.axis_index("c")
        s = lax.axis_index("s")
        wid = s * NC + c

        def fill(i, _):
            zv[pl.ds(i * 16, 16)] = jnp.zeros((16,), jnp.float32)
            return 0

        lax.fori_loop(0, RPT // 16, fill, 0)
        for i in range(K // 16):
            onesv[pl.ds(i * 16, 16)] = jnp.ones((16,), jnp.float32)
        pltpu.sync_copy(zv, acc_sh.at[pl.ds(s * RPT, RPT)])
        pltpu.sync_copy(dst_hbm.at[wid], dstv)
        plsc.subcore_barrier()

        def body(j, _):
            pltpu.sync_copy(onesv, acc_sh.at[dstv.at[j]], add=True)
            return 0

        lax.fori_loop(0, CH, body, 0)
        plsc.subcore_barrier()
        pltpu.sync_copy(acc_sh.at[pl.ds(s * RPT, RPT)],
                        out_hbm.at[c, pl.ds(s * RPT, RPT)])

    return deg_kernel(dst_r)


def _sc_scatter(hs, src_r, dst_r, d, tc_tiling=True):
    @functools.partial(
        pl.kernel,
        out_type=jax.ShapeDtypeStruct((NC, NPAD, d), jnp.float32),
        mesh=_mesh(),
        compiler_params=pltpu.CompilerParams(use_tc_tiling_on_sc=tc_tiling),
        scratch_types=[
            pltpu.VMEM((2, IB, K), jnp.int32),
            pltpu.VMEM((2, IB, K), jnp.int32),
            pltpu.VMEM((K, d), jnp.float32),
            pltpu.VMEM((K, d), jnp.float32),
            pltpu.VMEM((K, d), jnp.float32),
            pltpu.VMEM_SHARED((NPAD, d), jnp.float32),
            pltpu.SemaphoreType.DMA,
            pltpu.SemaphoreType.DMA,
            pltpu.SemaphoreType.DMA,
            pltpu.SemaphoreType.DMA,
            pltpu.SemaphoreType.DMA,
            pltpu.SemaphoreType.DMA,
            pltpu.SemaphoreType.DMA,
        ],
    )
    def scat_kernel(hs_hbm, src_hbm, dst_hbm, out_hbm,
                    sib, dib, rows0, rows1, rows2, acc_sh,
                    semg0, semg1, semg2, semsc0, semsc1, semsc2, semi):
        c = lax.axis_index("c")
        s = lax.axis_index("s")
        wid = s * NC + c
        rows = (rows0, rows1, rows2)
        semg = (semg0, semg1, semg2)
        semsc = (semsc0, semsc1, semsc2)

        def zfill(r, _):
            for i in range(d // 16):
                rows0[r, pl.ds(i * 16, 16)] = jnp.zeros((16,), jnp.float32)
            return 0

        lax.fori_loop(0, K, zfill, 0)
        for i in range(RPT // K):
            pltpu.sync_copy(rows0, acc_sh.at[pl.ds(s * RPT + i * K, K)])
        pltpu.sync_copy(src_hbm.at[wid, 0], sib.at[0])
        pltpu.sync_copy(dst_hbm.at[wid, 0], dib.at[0])
        plsc.subcore_barrier()
        pltpu.async_copy(hs_hbm.at[sib.at[0, 0]], rows0, semg0)
        pltpu.async_copy(hs_hbm.at[sib.at[0, 1]], rows1, semg1)

        def body(b, _):
            p = lax.rem(b, 2)

            @pl.when(b < NB - 1)
            def _():
                pltpu.async_copy(src_hbm.at[wid, b + 1], sib.at[1 - p], semi)
                pltpu.async_copy(dst_hbm.at[wid, b + 1], dib.at[1 - p], semi)

            for ch in range(IB):
                m = ch % 3
                pltpu.make_async_copy(hs_hbm.at[sib.at[p, ch]],
                                      rows[m], semg[m]).wait()
                pltpu.async_copy(rows[m], acc_sh.at[dib.at[p, ch]],
                                 semsc[m], add=True)
                if ch == 0:
                    @pl.when(b > 0)
                    def _():
                        pltpu.make_async_copy(
                            rows[(IB - 1) % 3],
                            acc_sh.at[dib.at[1 - p, IB - 1]],
                            semsc[(IB - 1) % 3]).wait()
                else:
                    pltpu.make_async_copy(
                        rows[(ch - 1) % 3], acc_sh.at[dib.at[p, ch - 1]],
                        semsc[(ch - 1) % 3]).wait()
                if ch + 2 < IB:
                    pltpu.async_copy(hs_hbm.at[sib.at[p, ch + 2]],
                                     rows[(ch + 2) % 3], semg[(ch + 2) % 3])
                elif ch + 2 == IB:
                    @pl.when(b < NB - 1)
                    def _():
                        pltpu.make_async_copy(src_hbm.at[wid, b + 1],
                                              sib.at[1 - p], semi).wait()
                        pltpu.make_async_copy(dst_hbm.at[wid, b + 1],
                                              dib.at[1 - p], semi).wait()
                        pltpu.async_copy(hs_hbm.at[sib.at[1 - p, 0]],
                                         rows[IB % 3], semg[IB % 3])
                else:
                    @pl.when(b < NB - 1)
                    def _():
                        pltpu.async_copy(hs_hbm.at[sib.at[1 - p, 1]],
                                         rows[(IB + 1) % 3],
                                         semg[(IB + 1) % 3])
            return 0

        lax.fori_loop(0, NB, body, 0)
        pltpu.make_async_copy(rows[(IB - 1) % 3],
                              acc_sh.at[dib.at[(NB - 1) % 2, IB - 1]],
                              semsc[(IB - 1) % 3]).wait()
        plsc.subcore_barrier()
        pltpu.sync_copy(acc_sh.at[pl.ds(s * RPT, RPT)],
                        out_hbm.at[c, pl.ds(s * RPT, RPT)])

    return scat_kernel(hs, src_r, dst_r)


def _tc1_body(x_ref, w_ref, degt_ref, hs_ref, dinv_ref):
    deg = degt_ref[:N_NODES, 0:1] + degt_ref[:N_NODES, 1:2] + 1.0
    dinv = lax.rsqrt(deg)
    h = jnp.dot(x_ref[...], w_ref[...], preferred_element_type=jnp.float32)
    hs_ref[...] = dinv * h
    dinv_ref[...] = dinv


def _tc2_body(p_ref, hs1_ref, dinv_ref, b1_ref, w2_ref, hs2_ref):
    dinv = dinv_ref[...]
    z = dinv * (p_ref[0, :N_NODES, :] + p_ref[1, :N_NODES, :] + hs1_ref[...])
    z = z + b1_ref[...]
    a = _SELU_SCALE * jnp.where(z > 0, z, _SELU_ALPHA * (jnp.exp(z) - 1.0))
    h2 = jnp.dot(a, w2_ref[...], preferred_element_type=jnp.float32)
    hs2_ref[...] = dinv * h2


def _tc3_body(q_ref, hs2_ref, dinv_ref, b2_ref, out_ref):
    z = dinv_ref[...] * (q_ref[0, :N_NODES, :] + q_ref[1, :N_NODES, :]
                         + hs2_ref[...])
    out_ref[...] = z + b2_ref[...]


def kernel(x, edge_index, W1, b1, W2, b2):
    ei = edge_index.astype(jnp.int32)
    npadding = E_PAD - N_EDGES
    pad_src = (jnp.arange(npadding, dtype=jnp.int32) * 13) % N_NODES
    pad_dst = N_NODES + (jnp.arange(npadding, dtype=jnp.int32)
                         % (NPAD - N_NODES))
    src_full = jnp.concatenate([ei[0], pad_src])
    dst_full = jnp.concatenate([ei[1], pad_dst])
    dst_r = dst_full.reshape(NW, CH, K)
    src_rb = src_full.reshape(NW, NB, IB, K)
    dst_rb = dst_full.reshape(NW, NB, IB, K)

    deg_p = _sc_degree(dst_r)
    degt = deg_p.T

    hs1, dinv = pl.pallas_call(
        _tc1_body,
        out_shape=(jax.ShapeDtypeStruct((N_NODES, HID_DIM), jnp.float32),
                   jax.ShapeDtypeStruct((N_NODES, 1), jnp.float32)),
    )(x, W1, degt)

    p = _sc_scatter(hs1, src_rb, dst_rb, HID_DIM)

    hs2 = pl.pallas_call(
        _tc2_body,
        out_shape=jax.ShapeDtypeStruct((N_NODES, OUT_DIM), jnp.float32),
    )(p, hs1, dinv, b1.reshape(1, HID_DIM), W2)

    q = _sc_scatter(hs2, src_rb, dst_rb, OUT_DIM, tc_tiling=False)

    out = pl.pallas_call(
        _tc3_body,
        out_shape=jax.ShapeDtypeStruct((N_NODES, OUT_DIM), jnp.float32),
    )(q, hs2, dinv, b2.reshape(1, OUT_DIM))
    return out


# speedup vs baseline: 1.3236x; 1.0217x over previous
import functools

import jax
import jax.numpy as jnp
from jax import lax
from jax.experimental import pallas as pl
from jax.experimental.pallas import tpu as pltpu
from jax.experimental.pallas import tpu_sc as plsc

N_NODES = 10000
NPAD = 10240
IN_DIM = 128
HID_DIM = 128
OUT_DIM = 64
N_EDGES = 320000

NC = 2
NS = 16
NW = NC * NS
EPW = 10368
E_PAD = NW * EPW
RING = 4
K = 48
CH = 216
IB = 8
NB = CH // IB
RPT = NPAD // NS

_SELU_ALPHA = 1.6732632423543772
_SELU_SCALE = 1.0507009873554805


def _mesh():
    return plsc.VectorSubcoreMesh(core_axis_name="c", subcore_axis_name="s")


def _sc_degree(dst_r):
    @functools.partial(
        pl.kernel,
        out_type=jax.ShapeDtypeStruct((NC, NPAD), jnp.float32),
        mesh=_mesh(),
        scratch_types=[
            pltpu.VMEM((CH, K), jnp.int32),
            pltpu.VMEM((K,), jnp.float32),
            pltpu.VMEM((RPT,), jnp.float32),
            pltpu.VMEM_SHARED((NPAD,), jnp.float32),
        ],
    )
    def deg_kernel(dst_hbm, out_hbm, dstv, onesv, zv, acc_sh):
        c = lax.axis_index("c")
        s = lax.axis_index("s")
        wid = s * NC + c

        def fill(i, _):
            zv[pl.ds(i * 16, 16)] = jnp.zeros((16,), jnp.float32)
            return 0

        lax.fori_loop(0, RPT // 16, fill, 0)
        for i in range(K // 16):
            onesv[pl.ds(i * 16, 16)] = jnp.ones((16,), jnp.float32)
        pltpu.sync_copy(zv, acc_sh.at[pl.ds(s * RPT, RPT)])
        pltpu.sync_copy(dst_hbm.at[wid], dstv)
        plsc.subcore_barrier()

        def body(j, _):
            pltpu.sync_copy(onesv, acc_sh.at[dstv.at[j]], add=True)
            return 0

        lax.fori_loop(0, CH, body, 0)
        plsc.subcore_barrier()
        pltpu.sync_copy(acc_sh.at[pl.ds(s * RPT, RPT)],
                        out_hbm.at[c, pl.ds(s * RPT, RPT)])

    return deg_kernel(dst_r)


def _sc_scatter(hs, src_r, dst_r, d, tc_tiling=True):
    @functools.partial(
        pl.kernel,
        out_type=jax.ShapeDtypeStruct((NC, NPAD, d), jnp.float32),
        mesh=_mesh(),
        compiler_params=pltpu.CompilerParams(use_tc_tiling_on_sc=tc_tiling),
        scratch_types=(
            [pltpu.VMEM((2, IB, K), jnp.int32),
             pltpu.VMEM((2, IB, K), jnp.int32)]
            + [pltpu.VMEM((K, d), jnp.float32) for _ in range(RING)]
            + [pltpu.VMEM_SHARED((NPAD, d), jnp.float32)]
            + [pltpu.SemaphoreType.DMA for _ in range(2 * RING + 1)]
        ),
    )
    def scat_kernel(hs_hbm, src_hbm, dst_hbm, out_hbm, *scr):
        sib, dib = scr[0], scr[1]
        rows = scr[2:2 + RING]
        acc_sh = scr[2 + RING]
        semg = scr[3 + RING:3 + 2 * RING]
        semsc = scr[3 + 2 * RING:3 + 3 * RING]
        semi = scr[3 + 3 * RING]
        c = lax.axis_index("c")
        s = lax.axis_index("s")
        wid = s * NC + c

        def zfill(r, _):
            for i in range(d // 16):
                rows[0][r, pl.ds(i * 16, 16)] = jnp.zeros((16,), jnp.float32)
            return 0

        lax.fori_loop(0, K, zfill, 0)
        for i in range(RPT // K):
            pltpu.sync_copy(rows[0], acc_sh.at[pl.ds(s * RPT + i * K, K)])
        if RPT % K:
            pltpu.sync_copy(
                rows[0].at[pl.ds(0, RPT % K)],
                acc_sh.at[pl.ds(s * RPT + (RPT // K) * K, RPT % K)])
        pltpu.sync_copy(src_hbm.at[wid, 0], sib.at[0])
        pltpu.sync_copy(dst_hbm.at[wid, 0], dib.at[0])
        plsc.subcore_barrier()
        for i in range(RING - 1):
            pltpu.async_copy(hs_hbm.at[sib.at[0, i]], rows[i], semg[i])

        def body(b, _):
            p = lax.rem(b, 2)

            @pl.when(b < NB - 1)
            def _():
                pltpu.async_copy(src_hbm.at[wid, b + 1], sib.at[1 - p], semi)
                pltpu.async_copy(dst_hbm.at[wid, b + 1], dib.at[1 - p], semi)

            for ch in range(IB):
                m = ch % RING
                pltpu.make_async_copy(hs_hbm.at[sib.at[p, ch]],
                                      rows[m], semg[m]).wait()
                pltpu.async_copy(rows[m], acc_sh.at[dib.at[p, ch]],
                                 semsc[m], add=True)
                if ch == 0:
                    @pl.when(b > 0)
                    def _():
                        pltpu.make_async_copy(
                            rows[(IB - 1) % RING],
                            acc_sh.at[dib.at[1 - p, IB - 1]],
                            semsc[(IB - 1) % RING]).wait()
                else:
                    pltpu.make_async_copy(
                        rows[(ch - 1) % RING], acc_sh.at[dib.at[p, ch - 1]],
                        semsc[(ch - 1) % RING]).wait()
                la = ch + RING - 1
                if la < IB:
                    pltpu.async_copy(hs_hbm.at[sib.at[p, la]],
                                     rows[la % RING], semg[la % RING])
                elif la == IB:
                    @pl.when(b < NB - 1)
                    def _():
                        pltpu.make_async_copy(src_hbm.at[wid, b + 1],
                                              sib.at[1 - p], semi).wait()
                        pltpu.make_async_copy(dst_hbm.at[wid, b + 1],
                                              dib.at[1 - p], semi).wait()
                        pltpu.async_copy(hs_hbm.at[sib.at[1 - p, 0]],
                                         rows[la % RING], semg[la % RING])
                else:
                    @pl.when(b < NB - 1)
                    def _():
                        pltpu.async_copy(hs_hbm.at[sib.at[1 - p, la - IB]],
                                         rows[la % RING], semg[la % RING])
            return 0

        lax.fori_loop(0, NB, body, 0)
        pltpu.make_async_copy(rows[(IB - 1) % RING],
                              acc_sh.at[dib.at[(NB - 1) % 2, IB - 1]],
                              semsc[(IB - 1) % RING]).wait()
        plsc.subcore_barrier()
        pltpu.sync_copy(acc_sh.at[pl.ds(s * RPT, RPT)],
                        out_hbm.at[c, pl.ds(s * RPT, RPT)])

    return scat_kernel(hs, src_r, dst_r)


def _tc1_body(x_ref, w_ref, degt_ref, hs_ref, dinv_ref):
    deg = degt_ref[:N_NODES, 0:1] + degt_ref[:N_NODES, 1:2] + 1.0
    dinv = lax.rsqrt(deg)
    h = jnp.dot(x_ref[...], w_ref[...], preferred_element_type=jnp.float32)
    hs_ref[...] = dinv * h
    dinv_ref[...] = dinv


def _tc2_body(p_ref, hs1_ref, dinv_ref, b1_ref, w2_ref, hs2_ref):
    dinv = dinv_ref[...]
    z = dinv * (p_ref[0, :N_NODES, :] + p_ref[1, :N_NODES, :] + hs1_ref[...])
    z = z + b1_ref[...]
    a = _SELU_SCALE * jnp.where(z > 0, z, _SELU_ALPHA * (jnp.exp(z) - 1.0))
    h2 = jnp.dot(a, w2_ref[...], preferred_element_type=jnp.float32)
    hs2_ref[...] = dinv * h2


def _tc3_body(q_ref, hs2_ref, dinv_ref, b2_ref, out_ref):
    z = dinv_ref[...] * (q_ref[0, :N_NODES, :] + q_ref[1, :N_NODES, :]
                         + hs2_ref[...])
    out_ref[...] = z + b2_ref[...]


def kernel(x, edge_index, W1, b1, W2, b2):
    ei = edge_index.astype(jnp.int32)
    npadding = E_PAD - N_EDGES
    pad_src = (jnp.arange(npadding, dtype=jnp.int32) * 13) % N_NODES
    pad_dst = N_NODES + (jnp.arange(npadding, dtype=jnp.int32)
                         % (NPAD - N_NODES))
    src_full = jnp.concatenate([ei[0], pad_src])
    dst_full = jnp.concatenate([ei[1], pad_dst])
    dst_r = dst_full.reshape(NW, CH, K)
    src_rb = src_full.reshape(NW, NB, IB, K)
    dst_rb = dst_full.reshape(NW, NB, IB, K)

    deg_p = _sc_degree(dst_r)
    degt = deg_p.T

    hs1, dinv = pl.pallas_call(
        _tc1_body,
        out_shape=(jax.ShapeDtypeStruct((N_NODES, HID_DIM), jnp.float32),
                   jax.ShapeDtypeStruct((N_NODES, 1), jnp.float32)),
    )(x, W1, degt)

    p = _sc_scatter(hs1, src_rb, dst_rb, HID_DIM)

    hs2 = pl.pallas_call(
        _tc2_body,
        out_shape=jax.ShapeDtypeStruct((N_NODES, OUT_DIM), jnp.float32),
    )(p, hs1, dinv, b1.reshape(1, HID_DIM), W2)

    q = _sc_scatter(hs2, src_rb, dst_rb, OUT_DIM, tc_tiling=False)

    out = pl.pallas_call(
        _tc3_body,
        out_shape=jax.ShapeDtypeStruct((N_NODES, OUT_DIM), jnp.float32),
    )(q, hs2, dinv, b2.reshape(1, OUT_DIM))
    return out


# speedup vs baseline: 1.3916x; 1.0514x over previous
import functools

import numpy as np

import jax
import jax.numpy as jnp
from jax import lax
from jax.experimental import pallas as pl
from jax.experimental.pallas import tpu as pltpu
from jax.experimental.pallas import tpu_sc as plsc

N_NODES = 10000
NPAD = 10240
IN_DIM = 128
HID_DIM = 128
OUT_DIM = 64
N_EDGES = 320000

NC = 2
NS = 16
NW = NC * NS
EPW = 10368
E_PAD = NW * EPW
RING = 4
K = 48
CH = 216
IB = 8
NB = CH // IB
RPT = NPAD // NS

_SELU_ALPHA = 1.6732632423543772
_SELU_SCALE = 1.0507009873554805

_NPADDING = E_PAD - N_EDGES
_PAD_SRC = (np.arange(_NPADDING, dtype=np.int32) * 13) % N_NODES
_PAD_DST = (N_NODES
            + np.arange(_NPADDING, dtype=np.int32) % (NPAD - N_NODES))


def _mesh():
    return plsc.VectorSubcoreMesh(core_axis_name="c", subcore_axis_name="s")


KD = 128
CHD = EPW // KD


def _sc_degree(dst_r):
    @functools.partial(
        pl.kernel,
        out_type=jax.ShapeDtypeStruct((NC, NPAD), jnp.float32),
        mesh=_mesh(),
        scratch_types=[
            pltpu.VMEM((CHD, KD), jnp.int32),
            pltpu.VMEM((KD,), jnp.float32),
            pltpu.VMEM((RPT,), jnp.float32),
            pltpu.VMEM_SHARED((NPAD,), jnp.float32),
            pltpu.SemaphoreType.DMA,
            pltpu.SemaphoreType.DMA,
        ],
    )
    def deg_kernel(dst_hbm, out_hbm, dstv, onesv, zv, acc_sh, semA, semB):
        c = lax.axis_index("c")
        s = lax.axis_index("s")
        wid = s * NC + c

        def fill(i, _):
            zv[pl.ds(i * 16, 16)] = jnp.zeros((16,), jnp.float32)
            return 0

        lax.fori_loop(0, RPT // 16, fill, 0)
        for i in range(KD // 16):
            onesv[pl.ds(i * 16, 16)] = jnp.ones((16,), jnp.float32)
        pltpu.sync_copy(zv, acc_sh.at[pl.ds(s * RPT, RPT)])
        pltpu.sync_copy(dst_hbm.at[wid], dstv)
        plsc.subcore_barrier()
        pltpu.async_copy(onesv, acc_sh.at[dstv.at[0]], semA, add=True)

        def body(t, _):
            j = 2 * t
            pltpu.async_copy(onesv, acc_sh.at[dstv.at[j + 1]], semB, add=True)
            pltpu.make_async_copy(onesv, acc_sh.at[dstv.at[j]], semA).wait()
            pltpu.async_copy(onesv, acc_sh.at[dstv.at[j + 2]], semA, add=True)
            pltpu.make_async_copy(onesv, acc_sh.at[dstv.at[j + 1]],
                                  semB).wait()
            return 0

        lax.fori_loop(0, (CHD - 1) // 2, body, 0)
        pltpu.make_async_copy(onesv, acc_sh.at[dstv.at[CHD - 1]], semA).wait()
        plsc.subcore_barrier()
        pltpu.sync_copy(acc_sh.at[pl.ds(s * RPT, RPT)],
                        out_hbm.at[c, pl.ds(s * RPT, RPT)])

    return deg_kernel(dst_r)


def _sc_scatter(hs, src_r, dst_r, d, tc_tiling=True):
    @functools.partial(
        pl.kernel,
        out_type=jax.ShapeDtypeStruct((NC, NPAD, d), jnp.float32),
        mesh=_mesh(),
        compiler_params=pltpu.CompilerParams(use_tc_tiling_on_sc=tc_tiling),
        scratch_types=(
            [pltpu.VMEM((2, IB, K), jnp.int32),
             pltpu.VMEM((2, IB, K), jnp.int32)]
            + [pltpu.VMEM((K, d), jnp.float32) for _ in range(RING)]
            + [pltpu.VMEM_SHARED((NPAD, d), jnp.float32)]
            + [pltpu.SemaphoreType.DMA for _ in range(2 * RING + 1)]
        ),
    )
    def scat_kernel(hs_hbm, src_hbm, dst_hbm, out_hbm, *scr):
        sib, dib = scr[0], scr[1]
        rows = scr[2:2 + RING]
        acc_sh = scr[2 + RING]
        semg = scr[3 + RING:3 + 2 * RING]
        semsc = scr[3 + 2 * RING:3 + 3 * RING]
        semi = scr[3 + 3 * RING]
        c = lax.axis_index("c")
        s = lax.axis_index("s")
        wid = s * NC + c

        def zfill(r, _):
            for i in range(d // 16):
                rows[0][r, pl.ds(i * 16, 16)] = jnp.zeros((16,), jnp.float32)
            return 0

        lax.fori_loop(0, K, zfill, 0)
        for i in range(RPT // K):
            pltpu.sync_copy(rows[0], acc_sh.at[pl.ds(s * RPT + i * K, K)])
        if RPT % K:
            pltpu.sync_copy(
                rows[0].at[pl.ds(0, RPT % K)],
                acc_sh.at[pl.ds(s * RPT + (RPT // K) * K, RPT % K)])
        pltpu.sync_copy(src_hbm.at[wid, 0], sib.at[0])
        pltpu.sync_copy(dst_hbm.at[wid, 0], dib.at[0])
        plsc.subcore_barrier()
        for i in range(RING - 1):
            pltpu.async_copy(hs_hbm.at[sib.at[0, i]], rows[i], semg[i])

        def body(b, _):
            p = lax.rem(b, 2)

            @pl.when(b < NB - 1)
            def _():
                pltpu.async_copy(src_hbm.at[wid, b + 1], sib.at[1 - p], semi)
                pltpu.async_copy(dst_hbm.at[wid, b + 1], dib.at[1 - p], semi)

            for ch in range(IB):
                m = ch % RING
                pltpu.make_async_copy(hs_hbm.at[sib.at[p, ch]],
                                      rows[m], semg[m]).wait()
                pltpu.async_copy(rows[m], acc_sh.at[dib.at[p, ch]],
                                 semsc[m], add=True)
                if ch == 0:
                    @pl.when(b > 0)
                    def _():
                        pltpu.make_async_copy(
                            rows[(IB - 1) % RING],
                            acc_sh.at[dib.at[1 - p, IB - 1]],
                            semsc[(IB - 1) % RING]).wait()
                else:
                    pltpu.make_async_copy(
                        rows[(ch - 1) % RING], acc_sh.at[dib.at[p, ch - 1]],
                        semsc[(ch - 1) % RING]).wait()
                la = ch + RING - 1
                if la < IB:
                    pltpu.async_copy(hs_hbm.at[sib.at[p, la]],
                                     rows[la % RING], semg[la % RING])
                elif la == IB:
                    @pl.when(b < NB - 1)
                    def _():
                        pltpu.make_async_copy(src_hbm.at[wid, b + 1],
                                              sib.at[1 - p], semi).wait()
                        pltpu.make_async_copy(dst_hbm.at[wid, b + 1],
                                              dib.at[1 - p], semi).wait()
                        pltpu.async_copy(hs_hbm.at[sib.at[1 - p, 0]],
                                         rows[la % RING], semg[la % RING])
                else:
                    @pl.when(b < NB - 1)
                    def _():
                        pltpu.async_copy(hs_hbm.at[sib.at[1 - p, la - IB]],
                                         rows[la % RING], semg[la % RING])
            return 0

        lax.fori_loop(0, NB, body, 0)
        pltpu.make_async_copy(rows[(IB - 1) % RING],
                              acc_sh.at[dib.at[(NB - 1) % 2, IB - 1]],
                              semsc[(IB - 1) % RING]).wait()
        plsc.subcore_barrier()
        pltpu.sync_copy(acc_sh.at[pl.ds(s * RPT, RPT)],
                        out_hbm.at[c, pl.ds(s * RPT, RPT)])

    return scat_kernel(hs, src_r, dst_r)


def _tc1_body(x_ref, w_ref, degt_ref, hs_ref, dinv_ref):
    deg = degt_ref[:N_NODES, 0:1] + degt_ref[:N_NODES, 1:2] + 1.0
    dinv = lax.rsqrt(deg)
    h = jnp.dot(x_ref[...], w_ref[...], preferred_element_type=jnp.float32)
    hs_ref[...] = dinv * h
    dinv_ref[...] = dinv


def _tc2_body(p_ref, hs1_ref, dinv_ref, b1_ref, w2_ref, hs2_ref):
    dinv = dinv_ref[...]
    z = dinv * (p_ref[0, :N_NODES, :] + p_ref[1, :N_NODES, :] + hs1_ref[...])
    z = z + b1_ref[...]
    a = _SELU_SCALE * jnp.where(z > 0, z, _SELU_ALPHA * (jnp.exp(z) - 1.0))
    h2 = jnp.dot(a, w2_ref[...], preferred_element_type=jnp.float32)
    hs2_ref[...] = dinv * h2


def _tc3_body(q_ref, hs2_ref, dinv_ref, b2_ref, out_ref):
    z = dinv_ref[...] * (q_ref[0, :N_NODES, :] + q_ref[1, :N_NODES, :]
                         + hs2_ref[...])
    out_ref[...] = z + b2_ref[...]


def kernel(x, edge_index, W1, b1, W2, b2):
    ei = edge_index.astype(jnp.int32)
    src_full = jnp.concatenate([ei[0], jnp.asarray(_PAD_SRC)])
    dst_full = jnp.concatenate([ei[1], jnp.asarray(_PAD_DST)])
    dst_r = dst_full.reshape(NW, CHD, KD)
    src_rb = src_full.reshape(NW, NB, IB, K)
    dst_rb = dst_full.reshape(NW, NB, IB, K)

    deg_p = _sc_degree(dst_r)
    degt = deg_p.T

    hs1, dinv = pl.pallas_call(
        _tc1_body,
        out_shape=(jax.ShapeDtypeStruct((N_NODES, HID_DIM), jnp.float32),
                   jax.ShapeDtypeStruct((N_NODES, 1), jnp.float32)),
    )(x, W1, degt)

    p = _sc_scatter(hs1, src_rb, dst_rb, HID_DIM)

    hs2 = pl.pallas_call(
        _tc2_body,
        out_shape=jax.ShapeDtypeStruct((N_NODES, OUT_DIM), jnp.float32),
    )(p, hs1, dinv, b1.reshape(1, HID_DIM), W2)

    q = _sc_scatter(hs2, src_rb, dst_rb, OUT_DIM, tc_tiling=False)

    out = pl.pallas_call(
        _tc3_body,
        out_shape=jax.ShapeDtypeStruct((N_NODES, OUT_DIM), jnp.float32),
    )(q, hs2, dinv, b2.reshape(1, OUT_DIM))
    return out


# speedup vs baseline: 1.4079x; 1.0117x over previous
import functools

import numpy as np

import jax
import jax.numpy as jnp
from jax import lax
from jax.experimental import pallas as pl
from jax.experimental.pallas import tpu as pltpu
from jax.experimental.pallas import tpu_sc as plsc

N_NODES = 10000
NPAD = 10240
IN_DIM = 128
HID_DIM = 128
OUT_DIM = 64
N_EDGES = 320000

NC = 2
NS = 16
NW = NC * NS
EPW = 10368
E_PAD = NW * EPW
RING = 4
K = 48
CH = 216
IB = 8
NB = CH // IB
RPT = NPAD // NS

_SELU_ALPHA = 1.6732632423543772
_SELU_SCALE = 1.0507009873554805

_NPADDING = E_PAD - N_EDGES
_PAD_SRC = (np.arange(_NPADDING, dtype=np.int32) * 13) % N_NODES
_PAD_DST = (N_NODES
            + np.arange(_NPADDING, dtype=np.int32) % (NPAD - N_NODES))


def _mesh():
    return plsc.VectorSubcoreMesh(core_axis_name="c", subcore_axis_name="s")


KD = 128
CHD = EPW // KD


def _sc_degree(dst_r):
    @functools.partial(
        pl.kernel,
        out_type=jax.ShapeDtypeStruct((NC, NPAD), jnp.float32),
        mesh=_mesh(),
        scratch_types=[
            pltpu.VMEM((CHD, KD), jnp.int32),
            pltpu.VMEM((KD,), jnp.float32),
            pltpu.VMEM((RPT,), jnp.float32),
            pltpu.VMEM_SHARED((NPAD,), jnp.float32),
            pltpu.SemaphoreType.DMA,
            pltpu.SemaphoreType.DMA,
        ],
    )
    def deg_kernel(dst_hbm, out_hbm, dstv, onesv, zv, acc_sh, semA, semB):
        c = lax.axis_index("c")
        s = lax.axis_index("s")
        wid = s * NC + c

        def fill(i, _):
            zv[pl.ds(i * 16, 16)] = jnp.zeros((16,), jnp.float32)
            return 0

        lax.fori_loop(0, RPT // 16, fill, 0)
        for i in range(KD // 16):
            onesv[pl.ds(i * 16, 16)] = jnp.ones((16,), jnp.float32)
        pltpu.sync_copy(zv, acc_sh.at[pl.ds(s * RPT, RPT)])
        pltpu.sync_copy(dst_hbm.at[wid], dstv)
        plsc.subcore_barrier()
        pltpu.async_copy(onesv, acc_sh.at[dstv.at[0]], semA, add=True)

        def body(t, _):
            j = 2 * t
            pltpu.async_copy(onesv, acc_sh.at[dstv.at[j + 1]], semB, add=True)
            pltpu.make_async_copy(onesv, acc_sh.at[dstv.at[j]], semA).wait()
            pltpu.async_copy(onesv, acc_sh.at[dstv.at[j + 2]], semA, add=True)
            pltpu.make_async_copy(onesv, acc_sh.at[dstv.at[j + 1]],
                                  semB).wait()
            return 0

        lax.fori_loop(0, (CHD - 1) // 2, body, 0)
        pltpu.make_async_copy(onesv, acc_sh.at[dstv.at[CHD - 1]], semA).wait()
        plsc.subcore_barrier()
        pltpu.sync_copy(acc_sh.at[pl.ds(s * RPT, RPT)],
                        out_hbm.at[c, pl.ds(s * RPT, RPT)])

    return deg_kernel(dst_r)


def _sc_scatter(hs, src_r, dst_r, d, tc_tiling=True, stage=False):
    @functools.partial(
        pl.kernel,
        out_type=jax.ShapeDtypeStruct((NC, NPAD, d), jnp.float32),
        mesh=_mesh(),
        compiler_params=pltpu.CompilerParams(use_tc_tiling_on_sc=tc_tiling),
        scratch_types=(
            [pltpu.VMEM((2, IB, K), jnp.int32),
             pltpu.VMEM((2, IB, K), jnp.int32)]
            + [pltpu.VMEM((K, d), jnp.float32) for _ in range(RING)]
            + [pltpu.VMEM_SHARED((NPAD, d), jnp.float32)]
            + [pltpu.SemaphoreType.DMA for _ in range(2 * RING + 1)]
            + ([pltpu.VMEM_SHARED((N_NODES, d), jnp.float32)] if stage
               else [])
        ),
    )
    def scat_kernel(hs_hbm, src_hbm, dst_hbm, out_hbm, *scr):
        sib, dib = scr[0], scr[1]
        rows = scr[2:2 + RING]
        acc_sh = scr[2 + RING]
        semg = scr[3 + RING:3 + 2 * RING]
        semsc = scr[3 + 2 * RING:3 + 3 * RING]
        semi = scr[3 + 3 * RING]
        c = lax.axis_index("c")
        s = lax.axis_index("s")
        wid = s * NC + c
        if stage:
            table = scr[4 + 3 * RING]
            nrows = N_NODES // NS
            pltpu.sync_copy(hs_hbm.at[pl.ds(s * nrows, nrows)],
                            table.at[pl.ds(s * nrows, nrows)])
            gsrc = table
        else:
            gsrc = hs_hbm

        def zfill(r, _):
            for i in range(d // 16):
                rows[0][r, pl.ds(i * 16, 16)] = jnp.zeros((16,), jnp.float32)
            return 0

        lax.fori_loop(0, K, zfill, 0)
        for i in range(RPT // K):
            pltpu.sync_copy(rows[0], acc_sh.at[pl.ds(s * RPT + i * K, K)])
        if RPT % K:
            pltpu.sync_copy(
                rows[0].at[pl.ds(0, RPT % K)],
                acc_sh.at[pl.ds(s * RPT + (RPT // K) * K, RPT % K)])
        pltpu.sync_copy(src_hbm.at[wid, 0], sib.at[0])
        pltpu.sync_copy(dst_hbm.at[wid, 0], dib.at[0])
        plsc.subcore_barrier()
        for i in range(RING - 1):
            pltpu.async_copy(gsrc.at[sib.at[0, i]], rows[i], semg[i])

        def body(b, _):
            p = lax.rem(b, 2)

            @pl.when(b < NB - 1)
            def _():
                pltpu.async_copy(src_hbm.at[wid, b + 1], sib.at[1 - p], semi)
                pltpu.async_copy(dst_hbm.at[wid, b + 1], dib.at[1 - p], semi)

            for ch in range(IB):
                m = ch % RING
                pltpu.make_async_copy(gsrc.at[sib.at[p, ch]],
                                      rows[m], semg[m]).wait()
                pltpu.async_copy(rows[m], acc_sh.at[dib.at[p, ch]],
                                 semsc[m], add=True)
                if ch == 0:
                    @pl.when(b > 0)
                    def _():
                        pltpu.make_async_copy(
                            rows[(IB - 1) % RING],
                            acc_sh.at[dib.at[1 - p, IB - 1]],
                            semsc[(IB - 1) % RING]).wait()
                else:
                    pltpu.make_async_copy(
                        rows[(ch - 1) % RING], acc_sh.at[dib.at[p, ch - 1]],
                        semsc[(ch - 1) % RING]).wait()
                la = ch + RING - 1
                if la < IB:
                    pltpu.async_copy(gsrc.at[sib.at[p, la]],
                                     rows[la % RING], semg[la % RING])
                elif la == IB:
                    @pl.when(b < NB - 1)
                    def _():
                        pltpu.make_async_copy(src_hbm.at[wid, b + 1],
                                              sib.at[1 - p], semi).wait()
                        pltpu.make_async_copy(dst_hbm.at[wid, b + 1],
                                              dib.at[1 - p], semi).wait()
                        pltpu.async_copy(gsrc.at[sib.at[1 - p, 0]],
                                         rows[la % RING], semg[la % RING])
                else:
                    @pl.when(b < NB - 1)
                    def _():
                        pltpu.async_copy(gsrc.at[sib.at[1 - p, la - IB]],
                                         rows[la % RING], semg[la % RING])
            return 0

        lax.fori_loop(0, NB, body, 0)
        pltpu.make_async_copy(rows[(IB - 1) % RING],
                              acc_sh.at[dib.at[(NB - 1) % 2, IB - 1]],
                              semsc[(IB - 1) % RING]).wait()
        plsc.subcore_barrier()
        pltpu.sync_copy(acc_sh.at[pl.ds(s * RPT, RPT)],
                        out_hbm.at[c, pl.ds(s * RPT, RPT)])

    return scat_kernel(hs, src_r, dst_r)


def _tc1_body(x_ref, w_ref, degt_ref, hs_ref, dinv_ref):
    deg = degt_ref[:N_NODES, 0:1] + degt_ref[:N_NODES, 1:2] + 1.0
    dinv = lax.rsqrt(deg)
    h = jnp.dot(x_ref[...], w_ref[...], preferred_element_type=jnp.float32)
    hs_ref[...] = dinv * h
    dinv_ref[...] = dinv


def _tc2_body(p_ref, hs1_ref, dinv_ref, b1_ref, w2_ref, hs2_ref):
    dinv = dinv_ref[...]
    z = dinv * (p_ref[0, :N_NODES, :] + p_ref[1, :N_NODES, :] + hs1_ref[...])
    z = z + b1_ref[...]
    a = _SELU_SCALE * jnp.where(z > 0, z, _SELU_ALPHA * (jnp.exp(z) - 1.0))
    h2 = jnp.dot(a, w2_ref[...], preferred_element_type=jnp.float32)
    hs2_ref[...] = dinv * h2


def _tc3_body(q_ref, hs2_ref, dinv_ref, b2_ref, out_ref):
    z = dinv_ref[...] * (q_ref[0, :N_NODES, :] + q_ref[1, :N_NODES, :]
                         + hs2_ref[...])
    out_ref[...] = z + b2_ref[...]


def kernel(x, edge_index, W1, b1, W2, b2):
    ei = edge_index.astype(jnp.int32)
    src_full = jnp.concatenate([ei[0], jnp.asarray(_PAD_SRC)])
    dst_full = jnp.concatenate([ei[1], jnp.asarray(_PAD_DST)])
    dst_r = dst_full.reshape(NW, CHD, KD)
    src_rb = src_full.reshape(NW, NB, IB, K)
    dst_rb = dst_full.reshape(NW, NB, IB, K)

    deg_p = _sc_degree(dst_r)
    degt = deg_p.T

    hs1, dinv = pl.pallas_call(
        _tc1_body,
        out_shape=(jax.ShapeDtypeStruct((N_NODES, HID_DIM), jnp.float32),
                   jax.ShapeDtypeStruct((N_NODES, 1), jnp.float32)),
    )(x, W1, degt)

    p = _sc_scatter(hs1, src_rb, dst_rb, HID_DIM)

    hs2 = pl.pallas_call(
        _tc2_body,
        out_shape=jax.ShapeDtypeStruct((N_NODES, OUT_DIM), jnp.float32),
    )(p, hs1, dinv, b1.reshape(1, HID_DIM), W2)

    q = _sc_scatter(hs2, src_rb, dst_rb, OUT_DIM, tc_tiling=False,
                    stage=True)

    out = pl.pallas_call(
        _tc3_body,
        out_shape=jax.ShapeDtypeStruct((N_NODES, OUT_DIM), jnp.float32),
    )(q, hs2, dinv, b2.reshape(1, OUT_DIM))
    return out
